# baseline (speedup 1.0000x reference)
import functools

import jax
import jax.numpy as jnp
from jax import lax
from jax.experimental import pallas as pl
from jax.experimental.pallas import tpu as pltpu
from jax.experimental.pallas import tpu_sc as plsc

B = 16384
D = 64
NUM_CORES = 2
NUM_SUBCORES = 16
NW = NUM_CORES * NUM_SUBCORES
ROWS_PER_W = B // NW
CHUNK = 128
NCHUNK = ROWS_PER_W // CHUNK
L = 16


def _sc_scores_kernel(subj_h, obj_h, orel_h, srel_h, nsub_h, nobj_h,
                      argt_h, relt_h, out_h,
                      si_v, oi_v, ri_v, nri_v, nsi_v, noi_v,
                      s_b, o_b, r_b, nr_b, ns_b, no_b,
                      pos_v, negr_v, negs_v, nego_v, sem):
    wid = lax.axis_index("s") * NUM_CORES + lax.axis_index("c")
    base = wid * ROWS_PER_W

    pltpu.sync_copy(subj_h.at[pl.ds(base, ROWS_PER_W)], si_v)
    pltpu.sync_copy(obj_h.at[pl.ds(base, ROWS_PER_W)], oi_v)
    pltpu.sync_copy(orel_h.at[pl.ds(base, ROWS_PER_W)], ri_v)
    pltpu.sync_copy(srel_h.at[pl.ds(base, ROWS_PER_W)], nri_v)
    pltpu.sync_copy(nsub_h.at[pl.ds(base, ROWS_PER_W)], nsi_v)
    pltpu.sync_copy(nobj_h.at[pl.ds(base, ROWS_PER_W)], noi_v)

    for ci in range(NCHUNK):
        off = ci * CHUNK
        cps = [
            pltpu.async_copy(argt_h.at[si_v.at[pl.ds(off, CHUNK)]], s_b, sem),
            pltpu.async_copy(argt_h.at[oi_v.at[pl.ds(off, CHUNK)]], o_b, sem),
            pltpu.async_copy(relt_h.at[ri_v.at[pl.ds(off, CHUNK)]], r_b, sem),
            pltpu.async_copy(relt_h.at[nri_v.at[pl.ds(off, CHUNK)]], nr_b, sem),
            pltpu.async_copy(argt_h.at[nsi_v.at[pl.ds(off, CHUNK)]], ns_b, sem),
            pltpu.async_copy(argt_h.at[noi_v.at[pl.ds(off, CHUNK)]], no_b, sem),
        ]
        for cp in cps:
            cp.wait()

        def block_body(b, _, off=off):
            rb = b * L
            rows = rb + lax.iota(jnp.int32, L)

            def d_body(d, accs):
                pos, negr, negs, nego = accs
                cols = jnp.zeros((L,), jnp.int32) + d
                s = plsc.load_gather(s_b, [rows, cols])
                o = plsc.load_gather(o_b, [rows, cols])
                r = plsc.load_gather(r_b, [rows, cols])
                nr = plsc.load_gather(nr_b, [rows, cols])
                ns = plsc.load_gather(ns_b, [rows, cols])
                no = plsc.load_gather(no_b, [rows, cols])
                pred = s * o
                pos = pos + pred * r
                negr = negr + pred * nr
                negs = negs + (ns * o) * r
                nego = nego + (s * no) * r
                return (pos, negr, negs, nego)

            z = jnp.zeros((L,), jnp.float32)
            pos, negr, negs, nego = lax.fori_loop(0, D, d_body, (z, z, z, z))
            pos_v[pl.ds(off + rb, L)] = pos
            negr_v[pl.ds(off + rb, L)] = -negr
            negs_v[pl.ds(off + rb, L)] = -negs
            nego_v[pl.ds(off + rb, L)] = -nego
            return 0

        lax.fori_loop(0, CHUNK // L, block_body, 0)

    pltpu.sync_copy(pos_v, out_h.at[pl.ds(0 * B + base, ROWS_PER_W)])
    pltpu.sync_copy(negr_v, out_h.at[pl.ds(1 * B + base, ROWS_PER_W)])
    pltpu.sync_copy(negs_v, out_h.at[pl.ds(2 * B + base, ROWS_PER_W)])
    pltpu.sync_copy(nego_v, out_h.at[pl.ds(3 * B + base, ROWS_PER_W)])


def _sc_scores(subjects, objects, obs_rel, samp_rel, samp_subj, samp_obj,
               arg_table, rel_table):
    mesh = plsc.VectorSubcoreMesh(core_axis_name="c", subcore_axis_name="s",
                                  num_cores=NUM_CORES,
                                  num_subcores=NUM_SUBCORES)
    run = functools.partial(
        pl.kernel,
        out_type=jax.ShapeDtypeStruct((4 * B,), jnp.float32),
        mesh=mesh,
        compiler_params=pltpu.CompilerParams(needs_layout_passes=False,
                                             use_tc_tiling_on_sc=False),
        scratch_types=[
            pltpu.VMEM((ROWS_PER_W,), jnp.int32),
            pltpu.VMEM((ROWS_PER_W,), jnp.int32),
            pltpu.VMEM((ROWS_PER_W,), jnp.int32),
            pltpu.VMEM((ROWS_PER_W,), jnp.int32),
            pltpu.VMEM((ROWS_PER_W,), jnp.int32),
            pltpu.VMEM((ROWS_PER_W,), jnp.int32),
            pltpu.VMEM((CHUNK, D), jnp.float32),
            pltpu.VMEM((CHUNK, D), jnp.float32),
            pltpu.VMEM((CHUNK, D), jnp.float32),
            pltpu.VMEM((CHUNK, D), jnp.float32),
            pltpu.VMEM((CHUNK, D), jnp.float32),
            pltpu.VMEM((CHUNK, D), jnp.float32),
            pltpu.VMEM((ROWS_PER_W,), jnp.float32),
            pltpu.VMEM((ROWS_PER_W,), jnp.float32),
            pltpu.VMEM((ROWS_PER_W,), jnp.float32),
            pltpu.VMEM((ROWS_PER_W,), jnp.float32),
            pltpu.SemaphoreType.DMA,
        ],
    )(_sc_scores_kernel)
    return run(subjects, objects, obs_rel, samp_rel, samp_subj, samp_obj,
               arg_table, rel_table)


def _tc_loss_kernel(x_ref, o_ref):
    y = x_ref[...]
    ls = jnp.minimum(y, 0.0) - jnp.log1p(jnp.exp(-jnp.abs(y)))
    o_ref[0, 0] = -jnp.sum(ls) * (1.0 / B)


def _tc_loss(scores):
    x = scores.reshape(4 * B // 128, 128)
    out = pl.pallas_call(
        _tc_loss_kernel,
        out_shape=jax.ShapeDtypeStruct((1, 1), jnp.float32),
        out_specs=pl.BlockSpec(memory_space=pltpu.SMEM),
    )(x)
    return out[0, 0]


def kernel(subjects, objects, observed_relations, sampled_relations,
           sampled_subjects, sampled_objects, arg_table, rel_table):
    i32 = jnp.int32
    scores = _sc_scores(
        subjects.astype(i32),
        objects.astype(i32),
        observed_relations[:, 0].astype(i32),
        sampled_relations[:, 0].astype(i32),
        sampled_subjects.astype(i32),
        sampled_objects.astype(i32),
        arg_table,
        rel_table,
    )
    return _tc_loss(scores)


# speedup vs baseline: 1.0111x; 1.0111x over previous
import functools

import jax
import jax.numpy as jnp
from jax import lax
from jax.experimental import pallas as pl
from jax.experimental.pallas import tpu as pltpu
from jax.experimental.pallas import tpu_sc as plsc

B = 16384
D = 64
NUM_CORES = 2
NUM_SUBCORES = 16
NW = NUM_CORES * NUM_SUBCORES
ROWS_PER_W = B // NW
CHUNK = 128
NCHUNK = ROWS_PER_W // CHUNK
L = 16
UNROLL = 4


def _sc_scores_kernel(subj_h, obj_h, orel_h, srel_h, nsub_h, nobj_h,
                      argt_h, relt_h, out_h,
                      si_v, oi_v, ri_v, nri_v, nsi_v, noi_v,
                      s_b0, o_b0, r_b0, nr_b0, ns_b0, no_b0,
                      s_b1, o_b1, r_b1, nr_b1, ns_b1, no_b1,
                      pos_v, negr_v, negs_v, nego_v, sem0, sem1):
    wid = lax.axis_index("s") * NUM_CORES + lax.axis_index("c")
    base = wid * ROWS_PER_W

    pltpu.sync_copy(subj_h.at[pl.ds(base, ROWS_PER_W)], si_v)
    pltpu.sync_copy(obj_h.at[pl.ds(base, ROWS_PER_W)], oi_v)
    pltpu.sync_copy(orel_h.at[pl.ds(base, ROWS_PER_W)], ri_v)
    pltpu.sync_copy(srel_h.at[pl.ds(base, ROWS_PER_W)], nri_v)
    pltpu.sync_copy(nsub_h.at[pl.ds(base, ROWS_PER_W)], nsi_v)
    pltpu.sync_copy(nobj_h.at[pl.ds(base, ROWS_PER_W)], noi_v)

    bufs = [(s_b0, o_b0, r_b0, nr_b0, ns_b0, no_b0),
            (s_b1, o_b1, r_b1, nr_b1, ns_b1, no_b1)]
    sems = [sem0, sem1]

    def fire(ci):
        off = ci * CHUNK
        sb, ob, rb_, nrb, nsb, nob = bufs[ci % 2]
        sm = sems[ci % 2]
        return [
            pltpu.async_copy(argt_h.at[si_v.at[pl.ds(off, CHUNK)]], sb, sm),
            pltpu.async_copy(argt_h.at[oi_v.at[pl.ds(off, CHUNK)]], ob, sm),
            pltpu.async_copy(relt_h.at[ri_v.at[pl.ds(off, CHUNK)]], rb_, sm),
            pltpu.async_copy(relt_h.at[nri_v.at[pl.ds(off, CHUNK)]], nrb, sm),
            pltpu.async_copy(argt_h.at[nsi_v.at[pl.ds(off, CHUNK)]], nsb, sm),
            pltpu.async_copy(argt_h.at[noi_v.at[pl.ds(off, CHUNK)]], nob, sm),
        ]

    pending = {0: fire(0)}
    for ci in range(NCHUNK):
        off = ci * CHUNK
        if ci + 1 < NCHUNK:
            pending[ci + 1] = fire(ci + 1)
        for cp in pending.pop(ci):
            cp.wait()
        sb, ob, rb_, nrb, nsb, nob = bufs[ci % 2]

        def block_body(b, _, off=off, sb=sb, ob=ob, rb_=rb_, nrb=nrb,
                       nsb=nsb, nob=nob):
            rbase = b * L
            rows = rbase + lax.iota(jnp.int32, L)

            def d_body(dq, accs):
                pos, negr, negs, nego = accs
                for u in range(UNROLL):
                    d = dq * UNROLL + u
                    cols = jnp.zeros((L,), jnp.int32) + d
                    s = plsc.load_gather(sb, [rows, cols])
                    o = plsc.load_gather(ob, [rows, cols])
                    r = plsc.load_gather(rb_, [rows, cols])
                    nr = plsc.load_gather(nrb, [rows, cols])
                    ns = plsc.load_gather(nsb, [rows, cols])
                    no = plsc.load_gather(nob, [rows, cols])
                    pred = s * o
                    pos = pos + pred * r
                    negr = negr + pred * nr
                    negs = negs + (ns * o) * r
                    nego = nego + (s * no) * r
                return (pos, negr, negs, nego)

            z = jnp.zeros((L,), jnp.float32)
            pos, negr, negs, nego = lax.fori_loop(0, D // UNROLL, d_body,
                                                  (z, z, z, z))
            pos_v[pl.ds(off + rbase, L)] = pos
            negr_v[pl.ds(off + rbase, L)] = -negr
            negs_v[pl.ds(off + rbase, L)] = -negs
            nego_v[pl.ds(off + rbase, L)] = -nego
            return 0

        lax.fori_loop(0, CHUNK // L, block_body, 0)

    pltpu.sync_copy(pos_v, out_h.at[pl.ds(0 * B + base, ROWS_PER_W)])
    pltpu.sync_copy(negr_v, out_h.at[pl.ds(1 * B + base, ROWS_PER_W)])
    pltpu.sync_copy(negs_v, out_h.at[pl.ds(2 * B + base, ROWS_PER_W)])
    pltpu.sync_copy(nego_v, out_h.at[pl.ds(3 * B + base, ROWS_PER_W)])


def _sc_scores(subjects, objects, obs_rel, samp_rel, samp_subj, samp_obj,
               arg_table, rel_table):
    mesh = plsc.VectorSubcoreMesh(core_axis_name="c", subcore_axis_name="s",
                                  num_cores=NUM_CORES,
                                  num_subcores=NUM_SUBCORES)
    run = functools.partial(
        pl.kernel,
        out_type=jax.ShapeDtypeStruct((4 * B,), jnp.float32),
        mesh=mesh,
        compiler_params=pltpu.CompilerParams(needs_layout_passes=False,
                                             use_tc_tiling_on_sc=False),
        scratch_types=[
            pltpu.VMEM((ROWS_PER_W,), jnp.int32),
            pltpu.VMEM((ROWS_PER_W,), jnp.int32),
            pltpu.VMEM((ROWS_PER_W,), jnp.int32),
            pltpu.VMEM((ROWS_PER_W,), jnp.int32),
            pltpu.VMEM((ROWS_PER_W,), jnp.int32),
            pltpu.VMEM((ROWS_PER_W,), jnp.int32),
            pltpu.VMEM((CHUNK, D), jnp.float32),
            pltpu.VMEM((CHUNK, D), jnp.float32),
            pltpu.VMEM((CHUNK, D), jnp.float32),
            pltpu.VMEM((CHUNK, D), jnp.float32),
            pltpu.VMEM((CHUNK, D), jnp.float32),
            pltpu.VMEM((CHUNK, D), jnp.float32),
            pltpu.VMEM((CHUNK, D), jnp.float32),
            pltpu.VMEM((CHUNK, D), jnp.float32),
            pltpu.VMEM((CHUNK, D), jnp.float32),
            pltpu.VMEM((CHUNK, D), jnp.float32),
            pltpu.VMEM((CHUNK, D), jnp.float32),
            pltpu.VMEM((CHUNK, D), jnp.float32),
            pltpu.VMEM((ROWS_PER_W,), jnp.float32),
            pltpu.VMEM((ROWS_PER_W,), jnp.float32),
            pltpu.VMEM((ROWS_PER_W,), jnp.float32),
            pltpu.VMEM((ROWS_PER_W,), jnp.float32),
            pltpu.SemaphoreType.DMA,
            pltpu.SemaphoreType.DMA,
        ],
    )(_sc_scores_kernel)
    return run(subjects, objects, obs_rel, samp_rel, samp_subj, samp_obj,
               arg_table, rel_table)


def _tc_loss_kernel(x_ref, o_ref):
    y = x_ref[...]
    ls = jnp.minimum(y, 0.0) - jnp.log1p(jnp.exp(-jnp.abs(y)))
    o_ref[0, 0] = -jnp.sum(ls) * (1.0 / B)


def _tc_loss(scores):
    x = scores.reshape(4 * B // 128, 128)
    out = pl.pallas_call(
        _tc_loss_kernel,
        out_shape=jax.ShapeDtypeStruct((1, 1), jnp.float32),
        out_specs=pl.BlockSpec(memory_space=pltpu.SMEM),
    )(x)
    return out[0, 0]


def kernel(subjects, objects, observed_relations, sampled_relations,
           sampled_subjects, sampled_objects, arg_table, rel_table):
    i32 = jnp.int32
    scores = _sc_scores(
        subjects.astype(i32),
        objects.astype(i32),
        observed_relations[:, 0].astype(i32),
        sampled_relations[:, 0].astype(i32),
        sampled_subjects.astype(i32),
        sampled_objects.astype(i32),
        arg_table,
        rel_table,
    )
    return _tc_loss(scores)


# speedup vs baseline: 1.0258x; 1.0145x over previous
import functools

import jax
import jax.numpy as jnp
from jax import lax
from jax.experimental import pallas as pl
from jax.experimental.pallas import tpu as pltpu
from jax.experimental.pallas import tpu_sc as plsc

B = 16384
D = 64
WIDE = 128
NUM_CORES = 2
NUM_SUBCORES = 16
NW = NUM_CORES * NUM_SUBCORES
ROWS_PER_W = B // NW
CHUNK = 64
NCHUNK = ROWS_PER_W // CHUNK
L = 16
UNROLL = 4


def _sc_scores_kernel(subj_h, obj_h, orel_h, srel_h, nsub_h, nobj_h,
                      subjp_h, objp_h, orelp_h, srelp_h, nsubp_h, nobjp_h,
                      argt_h, relt_h, out_h,
                      si_v, oi_v, ri_v, nri_v, nsi_v, noi_v,
                      sp_v, op_v, rp_v, nrp_v, nsp_v, nop_v,
                      s_b0, o_b0, r_b0, nr_b0, ns_b0, no_b0,
                      s_b1, o_b1, r_b1, nr_b1, ns_b1, no_b1,
                      pos_v, negr_v, negs_v, nego_v, sem0, sem1):
    wid = lax.axis_index("s") * NUM_CORES + lax.axis_index("c")
    base = wid * ROWS_PER_W

    pltpu.sync_copy(subj_h.at[pl.ds(base, ROWS_PER_W)], si_v)
    pltpu.sync_copy(obj_h.at[pl.ds(base, ROWS_PER_W)], oi_v)
    pltpu.sync_copy(orel_h.at[pl.ds(base, ROWS_PER_W)], ri_v)
    pltpu.sync_copy(srel_h.at[pl.ds(base, ROWS_PER_W)], nri_v)
    pltpu.sync_copy(nsub_h.at[pl.ds(base, ROWS_PER_W)], nsi_v)
    pltpu.sync_copy(nobj_h.at[pl.ds(base, ROWS_PER_W)], noi_v)
    pltpu.sync_copy(subjp_h.at[pl.ds(base, ROWS_PER_W)], sp_v)
    pltpu.sync_copy(objp_h.at[pl.ds(base, ROWS_PER_W)], op_v)
    pltpu.sync_copy(orelp_h.at[pl.ds(base, ROWS_PER_W)], rp_v)
    pltpu.sync_copy(srelp_h.at[pl.ds(base, ROWS_PER_W)], nrp_v)
    pltpu.sync_copy(nsubp_h.at[pl.ds(base, ROWS_PER_W)], nsp_v)
    pltpu.sync_copy(nobjp_h.at[pl.ds(base, ROWS_PER_W)], nop_v)

    bufs = [(s_b0, o_b0, r_b0, nr_b0, ns_b0, no_b0),
            (s_b1, o_b1, r_b1, nr_b1, ns_b1, no_b1)]
    sems = [sem0, sem1]

    def fire(ci):
        off = ci * CHUNK
        sb, ob, rb_, nrb, nsb, nob = bufs[ci % 2]
        sm = sems[ci % 2]
        return [
            pltpu.async_copy(argt_h.at[sp_v.at[pl.ds(off, CHUNK)]], sb, sm),
            pltpu.async_copy(argt_h.at[op_v.at[pl.ds(off, CHUNK)]], ob, sm),
            pltpu.async_copy(relt_h.at[rp_v.at[pl.ds(off, CHUNK)]], rb_, sm),
            pltpu.async_copy(relt_h.at[nrp_v.at[pl.ds(off, CHUNK)]], nrb, sm),
            pltpu.async_copy(argt_h.at[nsp_v.at[pl.ds(off, CHUNK)]], nsb, sm),
            pltpu.async_copy(argt_h.at[nop_v.at[pl.ds(off, CHUNK)]], nob, sm),
        ]

    pending = {0: fire(0)}
    for ci in range(NCHUNK):
        off = ci * CHUNK
        if ci + 1 < NCHUNK:
            pending[ci + 1] = fire(ci + 1)
        for cp in pending.pop(ci):
            cp.wait()
        sb, ob, rb_, nrb, nsb, nob = bufs[ci % 2]

        def block_body(b, _, off=off, sb=sb, ob=ob, rb_=rb_, nrb=nrb,
                       nsb=nsb, nob=nob):
            rbase = b * L
            rows = rbase + lax.iota(jnp.int32, L)
            g = off + rbase
            c_s = (si_v[pl.ds(g, L)] & 1) * D
            c_o = (oi_v[pl.ds(g, L)] & 1) * D
            c_r = (ri_v[pl.ds(g, L)] & 1) * D
            c_nr = (nri_v[pl.ds(g, L)] & 1) * D
            c_ns = (nsi_v[pl.ds(g, L)] & 1) * D
            c_no = (noi_v[pl.ds(g, L)] & 1) * D

            def d_body(dq, accs):
                pos, negr, negs, nego = accs
                for u in range(UNROLL):
                    d = dq * UNROLL + u
                    s = plsc.load_gather(sb, [rows, c_s + d])
                    o = plsc.load_gather(ob, [rows, c_o + d])
                    r = plsc.load_gather(rb_, [rows, c_r + d])
                    nr = plsc.load_gather(nrb, [rows, c_nr + d])
                    ns = plsc.load_gather(nsb, [rows, c_ns + d])
                    no = plsc.load_gather(nob, [rows, c_no + d])
                    pred = s * o
                    pos = pos + pred * r
                    negr = negr + pred * nr
                    negs = negs + (ns * o) * r
                    nego = nego + (s * no) * r
                return (pos, negr, negs, nego)

            z = jnp.zeros((L,), jnp.float32)
            pos, negr, negs, nego = lax.fori_loop(0, D // UNROLL, d_body,
                                                  (z, z, z, z))
            pos_v[pl.ds(g, L)] = pos
            negr_v[pl.ds(g, L)] = -negr
            negs_v[pl.ds(g, L)] = -negs
            nego_v[pl.ds(g, L)] = -nego
            return 0

        lax.fori_loop(0, CHUNK // L, block_body, 0)

    pltpu.sync_copy(pos_v, out_h.at[pl.ds(0 * B + base, ROWS_PER_W)])
    pltpu.sync_copy(negr_v, out_h.at[pl.ds(1 * B + base, ROWS_PER_W)])
    pltpu.sync_copy(negs_v, out_h.at[pl.ds(2 * B + base, ROWS_PER_W)])
    pltpu.sync_copy(nego_v, out_h.at[pl.ds(3 * B + base, ROWS_PER_W)])


def _sc_scores(idx6, pair6, arg_table2, rel_table2):
    mesh = plsc.VectorSubcoreMesh(core_axis_name="c", subcore_axis_name="s",
                                  num_cores=NUM_CORES,
                                  num_subcores=NUM_SUBCORES)
    idx_scratch = [pltpu.VMEM((ROWS_PER_W,), jnp.int32) for _ in range(12)]
    row_scratch = [pltpu.VMEM((CHUNK, WIDE), jnp.float32) for _ in range(12)]
    score_scratch = [pltpu.VMEM((ROWS_PER_W,), jnp.float32) for _ in range(4)]
    run = functools.partial(
        pl.kernel,
        out_type=jax.ShapeDtypeStruct((4 * B,), jnp.float32),
        mesh=mesh,
        compiler_params=pltpu.CompilerParams(needs_layout_passes=False,
                                             use_tc_tiling_on_sc=True),
        scratch_types=idx_scratch + row_scratch + score_scratch
        + [pltpu.SemaphoreType.DMA, pltpu.SemaphoreType.DMA],
    )(_sc_scores_kernel)
    return run(*idx6, *pair6, arg_table2, rel_table2)


def _tc_loss_kernel(x_ref, o_ref):
    y = x_ref[...]
    ls = jnp.minimum(y, 0.0) - jnp.log1p(jnp.exp(-jnp.abs(y)))
    o_ref[0, 0] = -jnp.sum(ls) * (1.0 / B)


def _tc_loss(scores):
    x = scores.reshape(4 * B // 128, 128)
    out = pl.pallas_call(
        _tc_loss_kernel,
        out_shape=jax.ShapeDtypeStruct((1, 1), jnp.float32),
        out_specs=pl.BlockSpec(memory_space=pltpu.SMEM),
    )(x)
    return out[0, 0]


def kernel(subjects, objects, observed_relations, sampled_relations,
           sampled_subjects, sampled_objects, arg_table, rel_table):
    i32 = jnp.int32
    idx6 = (
        subjects.astype(i32),
        objects.astype(i32),
        observed_relations[:, 0].astype(i32),
        sampled_relations[:, 0].astype(i32),
        sampled_subjects.astype(i32),
        sampled_objects.astype(i32),
    )
    pair6 = tuple(ix >> 1 for ix in idx6)
    arg2 = arg_table.reshape(arg_table.shape[0] // 2, WIDE)
    rel2 = rel_table.reshape(rel_table.shape[0] // 2, WIDE)
    scores = _sc_scores(idx6, pair6, arg2, rel2)
    return _tc_loss(scores)
